# Initial kernel scaffold; baseline (speedup 1.0000x reference)
import jax
import jax.numpy as jnp
from jax.experimental import pallas as pl


def kernel(features, index, node_emb, edge_index, edge_emb, n_sample, W, a):
    raise NotImplementedError("write your pallas kernel here")


# speedup vs baseline: 7.1134x; 7.1134x over previous
import functools

import jax
import jax.numpy as jnp
from jax import lax
from jax.experimental import pallas as pl
from jax.experimental.pallas import tpu as pltpu
from jax.experimental.pallas import tpu_sc as plsc

N = 2048
NS = 128
D = 256
DE = 16
EV = 65536
ALPHA = 0.2

NTILES = 32
ROWS_PER_TILE = N // NTILES
BATCH = 16
NBATCH = ROWS_PER_TILE // BATCH


def _threefry2x32(k0, k1, x0, x1):
    ks2 = k0 ^ k1 ^ jnp.uint32(0x1BD11BDA)
    ks = (k0, k1, ks2)
    rot = ((13, 15, 26, 6), (17, 29, 16, 24))
    x0 = x0 + k0
    x1 = x1 + k1
    for i in range(5):
        for r in rot[i % 2]:
            x0 = x0 + x1
            x1 = (x1 << jnp.uint32(r)) | (x1 >> jnp.uint32(32 - r))
            x1 = x0 ^ x1
        x0 = x0 + ks[(i + 1) % 3]
        x1 = x1 + ks[(i + 2) % 3] + jnp.uint32(i + 1)
    return x0, x1


def _prep_body(ei_ref, feat_ref, w_ref, a12t_ref, embt_ref,
               x_ref, s12_ref, s3_ref, u_ref, probs_ref):
    x = jnp.dot(feat_ref[...], w_ref[...], preferred_element_type=jnp.float32)
    x_ref[...] = x
    s12_ref[...] = lax.dot_general(
        a12t_ref[...], x, (((1,), (1,)), ((), ())),
        preferred_element_type=jnp.float32)
    s3 = jnp.zeros((EV // 128, 128), jnp.float32)
    for d in range(DE):
        s3 = s3 + embt_ref[d] * a12t_ref[2, d]
    s3_ref[...] = s3

    zero = jnp.zeros((N, NS), jnp.uint32)
    i_mat = lax.broadcasted_iota(jnp.uint32, (N, NS), 0)
    j_mat = lax.broadcasted_iota(jnp.uint32, (N, NS), 1)
    k0, k1 = _threefry2x32(zero, jnp.full((N, NS), 42, jnp.uint32), zero, i_mat)
    a, b = _threefry2x32(k0, k1, zero, j_mat)
    bits = a ^ b
    fb = (bits >> jnp.uint32(9)) | jnp.uint32(0x3F800000)
    u = lax.bitcast_convert_type(fb, jnp.float32) - jnp.float32(1.0)
    u_ref[...] = jnp.maximum(u, jnp.float32(0.0))

    mask = (ei_ref[...] != 0).astype(jnp.float32)
    rowsum = jnp.sum(mask, axis=1, keepdims=True)
    rows = lax.broadcasted_iota(jnp.int32, (N, N), 0)
    cols = lax.broadcasted_iota(jnp.int32, (N, N), 1)
    eye = (rows == cols).astype(jnp.float32)
    probs_ref[...] = jnp.where(rowsum > 0.0,
                               mask / jnp.maximum(rowsum, 1.0), eye)


def _sc_body(ei_hbm, pc_hbm, u_hbm, s12_hbm, s3_hbm, zz_hbm,
             a_hbm,
             s3_v, s12_v, av, pc_v, er_v, u_v):
    wid = lax.axis_index("s") * 2 + lax.axis_index("c")
    pltpu.sync_copy(s3_hbm, s3_v)
    pltpu.sync_copy(s12_hbm, s12_v)
    zero16 = jnp.zeros((16,), jnp.int32)
    last16 = jnp.full((16,), N - 1, jnp.int32)

    def batch_body(bb, _):
        base = wid * ROWS_PER_TILE + bb * BATCH
        pltpu.sync_copy(zz_hbm, av)

        def row_body(t, _):
            i = base + t
            pltpu.sync_copy(pc_hbm.at[pl.ds(i * N, N)], pc_v)
            pltpu.sync_copy(ei_hbm.at[pl.ds(i * N, N)], er_v)
            pltpu.sync_copy(u_hbm.at[pl.ds(i * NS, NS)], u_v)
            i_s = jnp.full((16,), i, jnp.int32)
            total = plsc.load_gather(pc_v, [last16])
            s1 = plsc.load_gather(s12_v, [i_s])
            logits = []
            neighs = []
            for k in range(8):
                u_k = u_v[pl.ds(16 * k, 16)]
                r_k = total * (jnp.float32(1.0) - u_k)
                pos = zero16
                w = 1024
                while w >= 1:
                    cand = pos + w
                    val = plsc.load_gather(pc_v, [cand - 1])
                    pos = jnp.where(val < r_k, cand, pos)
                    w //= 2
                e_k = plsc.load_gather(er_v, [pos])
                s2g = plsc.load_gather(s12_v, [pos + N])
                s3g = plsc.load_gather(s3_v, [e_k])
                z = s1 + s2g + s3g
                logits.append(jnp.where(z >= 0.0, z, jnp.float32(ALPHA) * z))
                neighs.append(pos)
            m = logits[0]
            for k in range(1, 8):
                m = jnp.maximum(m, logits[k])
            mx = lax.reduce_max(m, (0,))
            exps = [jnp.exp(l - mx) for l in logits]
            sv = exps[0]
            for k in range(1, 8):
                sv = sv + exps[k]
            ssum = lax.reduce_sum(sv, (0,))
            t_s = jnp.full((16,), t * N, jnp.int32)
            for k in range(8):
                att = exps[k] / ssum
                plsc.addupdate_scatter(av, [t_s + neighs[k]], att)
            return 0

        lax.fori_loop(0, BATCH, row_body, 0)
        pltpu.sync_copy(av, a_hbm.at[pl.ds(base * N, BATCH * N)])
        return 0

    lax.fori_loop(0, NBATCH, batch_body, 0)


def _finish_body(a_ref, x_ref, o_ref):
    i = pl.program_id(0)
    h = jnp.dot(a_ref[...], x_ref[...], preferred_element_type=jnp.float32)
    xb = x_ref[pl.ds(i * 256, 256), :]
    z = h + xb
    o_ref[...] = jnp.where(z > 0.0, z, jnp.exp(jnp.minimum(z, 0.0)) - 1.0)


@jax.jit
def _run(features, edge_index, edge_emb, W, a):
    a12t = jnp.concatenate(
        [a[:D, 0].reshape(1, D), a[D:2 * D, 0].reshape(1, D),
         jnp.pad(a[2 * D:, 0], (0, D - DE)).reshape(1, D)], axis=0)
    embt = edge_emb.T.reshape(DE, EV // 128, 128)

    x, s12t, s3t, u, probs = pl.pallas_call(
        _prep_body,
        out_shape=(
            jax.ShapeDtypeStruct((N, D), jnp.float32),
            jax.ShapeDtypeStruct((3, N), jnp.float32),
            jax.ShapeDtypeStruct((EV // 128, 128), jnp.float32),
            jax.ShapeDtypeStruct((N, NS), jnp.float32),
            jax.ShapeDtypeStruct((N, N), jnp.float32),
        ),
    )(edge_index, features, W, a12t, embt)

    pc = jnp.cumsum(probs, axis=-1)

    zz = jnp.zeros((BATCH * N,), jnp.float32)
    mesh = plsc.VectorSubcoreMesh(core_axis_name="c", subcore_axis_name="s")
    amat_f = pl.kernel(
        _sc_body,
        mesh=mesh,
        compiler_params=pltpu.CompilerParams(needs_layout_passes=False),
        out_type=jax.ShapeDtypeStruct((N * N,), jnp.float32),
        scratch_types=[
            pltpu.VMEM((EV,), jnp.float32),
            pltpu.VMEM((2 * N,), jnp.float32),
            pltpu.VMEM((BATCH * N,), jnp.float32),
            pltpu.VMEM((N,), jnp.float32),
            pltpu.VMEM((N,), jnp.int32),
            pltpu.VMEM((NS,), jnp.float32),
        ],
    )(edge_index.reshape(-1), pc.reshape(-1), u.reshape(-1),
      s12t[:2].reshape(-1), s3t.reshape(-1), zz)
    amat = amat_f.reshape(N, N)

    out = pl.pallas_call(
        _finish_body,
        grid=(8,),
        in_specs=[
            pl.BlockSpec((256, N), lambda i: (i, 0)),
            pl.BlockSpec((N, D), lambda i: (0, 0)),
        ],
        out_specs=pl.BlockSpec((256, D), lambda i: (i, 0)),
        out_shape=jax.ShapeDtypeStruct((N, D), jnp.float32),
    )(amat, x)
    return out


def kernel(features, index, node_emb, edge_index, edge_emb, n_sample, W, a):
    out = _run(features, edge_index, edge_emb, W, a)
    return (out, edge_emb)


# speedup vs baseline: 7.3445x; 1.0325x over previous
import functools

import jax
import jax.numpy as jnp
from jax import lax
from jax.experimental import pallas as pl
from jax.experimental.pallas import tpu as pltpu
from jax.experimental.pallas import tpu_sc as plsc

N = 2048
NS = 128
D = 256
DE = 16
EV = 65536
ALPHA = 0.2

NTILES = 32
ROWS_PER_TILE = N // NTILES
BATCH = 16
NBATCH = ROWS_PER_TILE // BATCH


def _threefry2x32(k0, k1, x0, x1):
    ks2 = k0 ^ k1 ^ jnp.uint32(0x1BD11BDA)
    ks = (k0, k1, ks2)
    rot = ((13, 15, 26, 6), (17, 29, 16, 24))
    x0 = x0 + k0
    x1 = x1 + k1
    for i in range(5):
        for r in rot[i % 2]:
            x0 = x0 + x1
            x1 = (x1 << jnp.uint32(r)) | (x1 >> jnp.uint32(32 - r))
            x1 = x0 ^ x1
        x0 = x0 + ks[(i + 1) % 3]
        x1 = x1 + ks[(i + 2) % 3] + jnp.uint32(i + 1)
    return x0, x1


def _prep_body(ei_ref, feat_ref, w_ref, a12t_ref, embt_ref,
               x_ref, s12_ref, s3_ref, u_ref, probs_ref):
    x = jnp.dot(feat_ref[...], w_ref[...], preferred_element_type=jnp.float32)
    x_ref[...] = x
    s12_ref[...] = lax.dot_general(
        a12t_ref[...], x, (((1,), (1,)), ((), ())),
        preferred_element_type=jnp.float32)
    s3 = jnp.zeros((EV // 128, 128), jnp.float32)
    for d in range(DE):
        s3 = s3 + embt_ref[d] * a12t_ref[2, d]
    s3_ref[...] = s3

    zero = jnp.zeros((N, NS), jnp.uint32)
    i_mat = lax.broadcasted_iota(jnp.uint32, (N, NS), 0)
    j_mat = lax.broadcasted_iota(jnp.uint32, (N, NS), 1)
    k0, k1 = _threefry2x32(zero, jnp.full((N, NS), 42, jnp.uint32), zero, i_mat)
    a, b = _threefry2x32(k0, k1, zero, j_mat)
    bits = a ^ b
    fb = (bits >> jnp.uint32(9)) | jnp.uint32(0x3F800000)
    u = lax.bitcast_convert_type(fb, jnp.float32) - jnp.float32(1.0)
    u_ref[...] = jnp.maximum(u, jnp.float32(0.0))

    mask = (ei_ref[...] != 0).astype(jnp.float32)
    rowsum = jnp.sum(mask, axis=1, keepdims=True)
    rows = lax.broadcasted_iota(jnp.int32, (N, N), 0)
    cols = lax.broadcasted_iota(jnp.int32, (N, N), 1)
    eye = (rows == cols).astype(jnp.float32)
    probs_ref[...] = jnp.where(rowsum > 0.0,
                               mask / jnp.maximum(rowsum, 1.0), eye)


def _sc_body(ei_hbm, pc_hbm, u_hbm, s12_hbm, s3_hbm, zz_hbm,
             a_hbm,
             s3_v, s12_v, av, pc_v, er_v, u_v):
    wid = lax.axis_index("s") * 2 + lax.axis_index("c")
    pltpu.sync_copy(s3_hbm, s3_v)
    pltpu.sync_copy(s12_hbm, s12_v)
    zero16 = jnp.zeros((16,), jnp.int32)
    last16 = jnp.full((16,), N - 1, jnp.int32)

    zero16f = jnp.zeros((16,), jnp.int32)

    def batch_body(bb, _):
        base = wid * ROWS_PER_TILE + bb * BATCH
        pltpu.sync_copy(zz_hbm, av)

        def row_body(t, _):
            i = base + t
            pltpu.sync_copy(pc_hbm.at[pl.ds(i, 1), :], pc_v)
            pltpu.sync_copy(ei_hbm.at[pl.ds(i, 1), :], er_v)
            pltpu.sync_copy(u_hbm.at[pl.ds(i, 1), :], u_v)
            i_s = jnp.full((16,), i, jnp.int32)
            total = plsc.load_gather(pc_v, [zero16f, last16])
            s1 = plsc.load_gather(s12_v, [zero16f, i_s])
            one16 = jnp.full((16,), 1, jnp.int32)
            logits = []
            neighs = []
            for k in range(8):
                u_k = u_v[0, pl.ds(16 * k, 16)]
                r_k = total * (jnp.float32(1.0) - u_k)
                pos = zero16
                w = 1024
                while w >= 1:
                    cand = pos + w
                    val = plsc.load_gather(pc_v, [zero16f, cand - 1])
                    pos = jnp.where(val < r_k, cand, pos)
                    w //= 2
                e_k = plsc.load_gather(er_v, [zero16f, pos])
                s2g = plsc.load_gather(s12_v, [one16, pos])
                s3g = plsc.load_gather(s3_v, [e_k >> 7, e_k & 127])
                z = s1 + s2g + s3g
                logits.append(jnp.where(z >= 0.0, z, jnp.float32(ALPHA) * z))
                neighs.append(pos)
            m = logits[0]
            for k in range(1, 8):
                m = jnp.maximum(m, logits[k])
            mx = lax.reduce_max(m, (0,))
            exps = [jnp.exp(l - mx) for l in logits]
            sv = exps[0]
            for k in range(1, 8):
                sv = sv + exps[k]
            ssum = lax.reduce_sum(sv, (0,))
            t_s = jnp.full((16,), t, jnp.int32)
            for k in range(8):
                att = exps[k] / ssum
                plsc.addupdate_scatter(av, [t_s, neighs[k]], att)
            return 0

        lax.fori_loop(0, BATCH, row_body, 0)
        pltpu.sync_copy(av, a_hbm.at[pl.ds(base, BATCH), :])
        return 0

    lax.fori_loop(0, NBATCH, batch_body, 0)


def _finish_body(a_ref, x_ref, o_ref):
    i = pl.program_id(0)
    h = jnp.dot(a_ref[...], x_ref[...], preferred_element_type=jnp.float32)
    xb = x_ref[pl.ds(i * 256, 256), :]
    z = h + xb
    o_ref[...] = jnp.where(z > 0.0, z, jnp.exp(jnp.minimum(z, 0.0)) - 1.0)


@jax.jit
def _run(features, edge_index, edge_emb, W, a):
    a12t = jnp.concatenate(
        [a[:D, 0].reshape(1, D), a[D:2 * D, 0].reshape(1, D),
         jnp.pad(a[2 * D:, 0], (0, D - DE)).reshape(1, D)], axis=0)
    embt = edge_emb.T.reshape(DE, EV // 128, 128)

    x, s12t, s3t, u, probs = pl.pallas_call(
        _prep_body,
        out_shape=(
            jax.ShapeDtypeStruct((N, D), jnp.float32),
            jax.ShapeDtypeStruct((3, N), jnp.float32),
            jax.ShapeDtypeStruct((EV // 128, 128), jnp.float32),
            jax.ShapeDtypeStruct((N, NS), jnp.float32),
            jax.ShapeDtypeStruct((N, N), jnp.float32),
        ),
    )(edge_index, features, W, a12t, embt)

    pc = jnp.cumsum(probs, axis=-1)

    zz = jnp.zeros((BATCH, N), jnp.float32)
    mesh = plsc.VectorSubcoreMesh(core_axis_name="c", subcore_axis_name="s")
    amat = pl.kernel(
        _sc_body,
        mesh=mesh,
        compiler_params=pltpu.CompilerParams(needs_layout_passes=False),
        out_type=jax.ShapeDtypeStruct((N, N), jnp.float32),
        scratch_types=[
            pltpu.VMEM((EV // 128, 128), jnp.float32),
            pltpu.VMEM((2, N), jnp.float32),
            pltpu.VMEM((BATCH, N), jnp.float32),
            pltpu.VMEM((1, N), jnp.float32),
            pltpu.VMEM((1, N), jnp.int32),
            pltpu.VMEM((1, NS), jnp.float32),
        ],
    )(edge_index, pc, u, s12t[:2], s3t, zz)

    out = pl.pallas_call(
        _finish_body,
        grid=(8,),
        in_specs=[
            pl.BlockSpec((256, N), lambda i: (i, 0)),
            pl.BlockSpec((N, D), lambda i: (0, 0)),
        ],
        out_specs=pl.BlockSpec((256, D), lambda i: (i, 0)),
        out_shape=jax.ShapeDtypeStruct((N, D), jnp.float32),
    )(amat, x)
    return out


def kernel(features, index, node_emb, edge_index, edge_emb, n_sample, W, a):
    out = _run(features, edge_index, edge_emb, W, a)
    return (out, edge_emb)


# speedup vs baseline: 10.1452x; 1.3813x over previous
import functools

import jax
import jax.numpy as jnp
from jax import lax
from jax.experimental import pallas as pl
from jax.experimental.pallas import tpu as pltpu
from jax.experimental.pallas import tpu_sc as plsc

N = 2048
NS = 128
D = 256
DE = 16
EV = 65536
ALPHA = 0.2

NTILES = 32
ROWS_PER_TILE = N // NTILES
BATCH = 16
NBATCH = ROWS_PER_TILE // BATCH


def _threefry2x32(k0, k1, x0, x1):
    ks2 = k0 ^ k1 ^ jnp.uint32(0x1BD11BDA)
    ks = (k0, k1, ks2)
    rot = ((13, 15, 26, 6), (17, 29, 16, 24))
    x0 = x0 + k0
    x1 = x1 + k1
    for i in range(5):
        for r in rot[i % 2]:
            x0 = x0 + x1
            x1 = (x1 << jnp.uint32(r)) | (x1 >> jnp.uint32(32 - r))
            x1 = x0 ^ x1
        x0 = x0 + ks[(i + 1) % 3]
        x1 = x1 + ks[(i + 2) % 3] + jnp.uint32(i + 1)
    return x0, x1


def _prep_body(ei_ref, feat_ref, w_ref, a12t_ref, embt_ref,
               x_ref, s12_ref, s3_ref, u_ref, probs_ref):
    x = jnp.dot(feat_ref[...], w_ref[...], preferred_element_type=jnp.float32)
    x_ref[...] = x
    s12_ref[...] = lax.dot_general(
        a12t_ref[...], x, (((1,), (1,)), ((), ())),
        preferred_element_type=jnp.float32)
    s3 = jnp.zeros((EV // 128, 128), jnp.float32)
    for d in range(DE):
        s3 = s3 + embt_ref[d] * a12t_ref[2, d]
    s3_ref[...] = s3

    zero = jnp.zeros((N, NS), jnp.uint32)
    i_mat = lax.broadcasted_iota(jnp.uint32, (N, NS), 0)
    j_mat = lax.broadcasted_iota(jnp.uint32, (N, NS), 1)
    k0, k1 = _threefry2x32(zero, jnp.full((N, NS), 42, jnp.uint32), zero, i_mat)
    a, b = _threefry2x32(k0, k1, zero, j_mat)
    bits = a ^ b
    fb = (bits >> jnp.uint32(9)) | jnp.uint32(0x3F800000)
    u = lax.bitcast_convert_type(fb, jnp.float32) - jnp.float32(1.0)
    u_ref[...] = jnp.maximum(u, jnp.float32(0.0))

    mask = (ei_ref[...] != 0).astype(jnp.float32)
    rowsum = jnp.sum(mask, axis=1, keepdims=True)
    rows = lax.broadcasted_iota(jnp.int32, (N, N), 0)
    cols = lax.broadcasted_iota(jnp.int32, (N, N), 1)
    eye = (rows == cols).astype(jnp.float32)
    probs_ref[...] = jnp.where(rowsum > 0.0,
                               mask / jnp.maximum(rowsum, 1.0), eye)


def _sc_body(ei_hbm, pc_hbm, u_hbm, s12_hbm, s3_hbm, zz_hbm,
             a_hbm,
             s3_v, s12_v, av,
             pc_v0, pc_v1, er_v0, er_v1, u_v0, u_v1,
             sem_pc0, sem_pc1, sem_er0, sem_er1, sem_u0, sem_u1):
    wid = lax.axis_index("s") * 2 + lax.axis_index("c")
    pltpu.sync_copy(s3_hbm, s3_v)
    pltpu.sync_copy(s12_hbm, s12_v)
    zero16 = jnp.zeros((16,), jnp.int32)
    last16 = jnp.full((16,), N - 1, jnp.int32)
    one16 = jnp.full((16,), 1, jnp.int32)
    pc_b = (pc_v0, pc_v1)
    er_b = (er_v0, er_v1)
    u_b = (u_v0, u_v1)
    sems = ((sem_pc0, sem_er0, sem_u0), (sem_pc1, sem_er1, sem_u1))

    def start_row(i, par):
        sp, se, su = sems[par]
        return (pltpu.async_copy(pc_hbm.at[pl.ds(i, 1), :], pc_b[par], sp),
                pltpu.async_copy(ei_hbm.at[pl.ds(i, 1), :], er_b[par], se),
                pltpu.async_copy(u_hbm.at[pl.ds(i, 1), :], u_b[par], su))

    def batch_body(bb, _):
        base = wid * ROWS_PER_TILE + bb * BATCH
        handles = [None, None]
        handles[0] = start_row(base, 0)
        pltpu.sync_copy(zz_hbm, av)
        for t in range(BATCH):
            par = t % 2
            for h in handles[par]:
                h.wait()
            if t + 1 < BATCH:
                handles[(t + 1) % 2] = start_row(base + t + 1, (t + 1) % 2)
            pc_v, er_v, u_v = pc_b[par], er_b[par], u_b[par]
            i_s = jnp.full((16,), base + t, jnp.int32)
            total = plsc.load_gather(pc_v, [zero16, last16])
            s1 = plsc.load_gather(s12_v, [zero16, i_s])
            logits = []
            neighs = []
            for k in range(8):
                u_k = u_v[0, pl.ds(16 * k, 16)]
                r_k = total * (jnp.float32(1.0) - u_k)
                pos = zero16
                w = 1024
                while w >= 1:
                    cand = pos + w
                    val = plsc.load_gather(pc_v, [zero16, cand - 1])
                    pos = jnp.where(val < r_k, cand, pos)
                    w //= 2
                e_k = plsc.load_gather(er_v, [zero16, pos])
                s2g = plsc.load_gather(s12_v, [one16, pos])
                s3g = plsc.load_gather(s3_v, [e_k >> 7, e_k & 127])
                z = s1 + s2g + s3g
                logits.append(jnp.where(z >= 0.0, z, jnp.float32(ALPHA) * z))
                neighs.append(pos)
            m = logits[0]
            for k in range(1, 8):
                m = jnp.maximum(m, logits[k])
            mx = lax.reduce_max(m, (0,))
            exps = [jnp.exp(l - mx) for l in logits]
            sv = exps[0]
            for k in range(1, 8):
                sv = sv + exps[k]
            ssum = lax.reduce_sum(sv, (0,))
            t_s = jnp.full((16,), t, jnp.int32)
            for k in range(8):
                att = exps[k] / ssum
                plsc.addupdate_scatter(av, [t_s, neighs[k]], att)
        pltpu.sync_copy(av, a_hbm.at[pl.ds(base, BATCH), :])
        return 0

    lax.fori_loop(0, NBATCH, batch_body, 0)


def _finish_body(a_ref, x_ref, o_ref):
    i = pl.program_id(0)
    h = jnp.dot(a_ref[...], x_ref[...], preferred_element_type=jnp.float32)
    xb = x_ref[pl.ds(i * 256, 256), :]
    z = h + xb
    o_ref[...] = jnp.where(z > 0.0, z, jnp.exp(jnp.minimum(z, 0.0)) - 1.0)


@jax.jit
def _run(features, edge_index, edge_emb, W, a):
    a12t = jnp.concatenate(
        [a[:D, 0].reshape(1, D), a[D:2 * D, 0].reshape(1, D),
         jnp.pad(a[2 * D:, 0], (0, D - DE)).reshape(1, D)], axis=0)
    embt = edge_emb.T.reshape(DE, EV // 128, 128)

    x, s12t, s3t, u, probs = pl.pallas_call(
        _prep_body,
        out_shape=(
            jax.ShapeDtypeStruct((N, D), jnp.float32),
            jax.ShapeDtypeStruct((3, N), jnp.float32),
            jax.ShapeDtypeStruct((EV // 128, 128), jnp.float32),
            jax.ShapeDtypeStruct((N, NS), jnp.float32),
            jax.ShapeDtypeStruct((N, N), jnp.float32),
        ),
    )(edge_index, features, W, a12t, embt)

    pc = jnp.cumsum(probs, axis=-1)

    zz = jnp.zeros((BATCH, N), jnp.float32)
    mesh = plsc.VectorSubcoreMesh(core_axis_name="c", subcore_axis_name="s")
    amat = pl.kernel(
        _sc_body,
        mesh=mesh,
        compiler_params=pltpu.CompilerParams(needs_layout_passes=False),
        out_type=jax.ShapeDtypeStruct((N, N), jnp.float32),
        scratch_types=[
            pltpu.VMEM((EV // 128, 128), jnp.float32),
            pltpu.VMEM((2, N), jnp.float32),
            pltpu.VMEM((BATCH, N), jnp.float32),
            pltpu.VMEM((1, N), jnp.float32),
            pltpu.VMEM((1, N), jnp.float32),
            pltpu.VMEM((1, N), jnp.int32),
            pltpu.VMEM((1, N), jnp.int32),
            pltpu.VMEM((1, NS), jnp.float32),
            pltpu.VMEM((1, NS), jnp.float32),
            pltpu.SemaphoreType.DMA,
            pltpu.SemaphoreType.DMA,
            pltpu.SemaphoreType.DMA,
            pltpu.SemaphoreType.DMA,
            pltpu.SemaphoreType.DMA,
            pltpu.SemaphoreType.DMA,
        ],
    )(edge_index, pc, u, s12t[:2], s3t, zz)

    out = pl.pallas_call(
        _finish_body,
        grid=(8,),
        in_specs=[
            pl.BlockSpec((256, N), lambda i: (i, 0)),
            pl.BlockSpec((N, D), lambda i: (0, 0)),
        ],
        out_specs=pl.BlockSpec((256, D), lambda i: (i, 0)),
        out_shape=jax.ShapeDtypeStruct((N, D), jnp.float32),
    )(amat, x)
    return out


def kernel(features, index, node_emb, edge_index, edge_emb, n_sample, W, a):
    out = _run(features, edge_index, edge_emb, W, a)
    return (out, edge_emb)


# speedup vs baseline: 10.7671x; 1.0613x over previous
import jax
import jax.numpy as jnp
from jax import lax
from jax.experimental import pallas as pl
from jax.experimental.pallas import tpu as pltpu
from jax.experimental.pallas import tpu_sc as plsc

N = 2048
NS = 128
D = 256
DE = 16
EV = 65536
CT = N // 128
ALPHA = 0.2

NTILES = 32
ROWS_PER_TILE = N // NTILES
BATCH = 16
NBATCH = ROWS_PER_TILE // BATCH


def _threefry2x32(k0, k1, x0, x1):
    ks2 = k0 ^ k1 ^ jnp.uint32(0x1BD11BDA)
    ks = (k0, k1, ks2)
    rot = ((13, 15, 26, 6), (17, 29, 16, 24))
    x0 = x0 + k0
    x1 = x1 + k1
    for i in range(5):
        for r in rot[i % 2]:
            x0 = x0 + x1
            x1 = (x1 << jnp.uint32(r)) | (x1 >> jnp.uint32(32 - r))
            x1 = x0 ^ x1
        x0 = x0 + ks[(i + 1) % 3]
        x1 = x1 + ks[(i + 2) % 3] + jnp.uint32(i + 1)
    return x0, x1


def _prep_body(ei_ref, feat_ref, w_ref, a12t_ref, embt_ref,
               x_ref, s12_ref, s3_ref, u_ref, rs_ref):
    x = jnp.dot(feat_ref[...], w_ref[...], preferred_element_type=jnp.float32)
    x_ref[...] = x
    s12_ref[...] = lax.dot_general(
        a12t_ref[...], x, (((1,), (1,)), ((), ())),
        preferred_element_type=jnp.float32)
    s3 = jnp.zeros((EV // 128, 128), jnp.float32)
    for d in range(DE):
        s3 = s3 + embt_ref[d] * a12t_ref[2, d]
    s3_ref[...] = s3

    zero = jnp.zeros((N, NS), jnp.uint32)
    i_mat = lax.broadcasted_iota(jnp.uint32, (N, NS), 0)
    j_mat = lax.broadcasted_iota(jnp.uint32, (N, NS), 1)
    k0, k1 = _threefry2x32(zero, jnp.full((N, NS), 42, jnp.uint32), zero, i_mat)
    a, b = _threefry2x32(k0, k1, zero, j_mat)
    bits = a ^ b
    fb = (bits >> jnp.uint32(9)) | jnp.uint32(0x3F800000)
    u = lax.bitcast_convert_type(fb, jnp.float32) - jnp.float32(1.0)
    u_ref[...] = jnp.maximum(u, jnp.float32(0.0))

    mask = (ei_ref[...] != 0).astype(jnp.float32)
    rs_ref[...] = jnp.sum(mask, axis=1, keepdims=True)


def _strip_body(ei_ref, rs_ref, p3_ref, e3_ref):
    i = pl.program_id(0)
    eib = ei_ref[...]
    mask = (eib != 0).astype(jnp.float32)
    rows = lax.broadcasted_iota(jnp.int32, (256, N), 0) + i * 256
    cols = lax.broadcasted_iota(jnp.int32, (256, N), 1)
    eye = (rows == cols).astype(jnp.float32)
    rs = rs_ref[...]
    probs = jnp.where(rs > 0.0, mask / jnp.maximum(rs, 1.0), eye)
    p3_ref[...] = probs.reshape(256, CT, 128)
    e3_ref[...] = eib.reshape(256, CT, 128)


def _sc_body(ei_hbm, pc_hbm, u_hbm, s12_hbm, s3_hbm, zz_hbm,
             a_hbm,
             s3_v, s12_v, av,
             pc_v0, pc_v1, er_v0, er_v1, u_v0, u_v1,
             sem_pc0, sem_pc1, sem_er0, sem_er1, sem_u0, sem_u1):
    wid = lax.axis_index("s") * 2 + lax.axis_index("c")
    pltpu.sync_copy(s3_hbm, s3_v)
    pltpu.sync_copy(s12_hbm, s12_v)
    zero16 = jnp.zeros((16,), jnp.int32)
    last_hi = jnp.full((16,), CT - 1, jnp.int32)
    last_lo = jnp.full((16,), 127, jnp.int32)
    one16 = jnp.full((16,), 1, jnp.int32)
    pc_b = (pc_v0, pc_v1)
    er_b = (er_v0, er_v1)
    u_b = (u_v0, u_v1)
    sems = ((sem_pc0, sem_er0, sem_u0), (sem_pc1, sem_er1, sem_u1))

    def start_row(i, par):
        sp, se, su = sems[par]
        return (pltpu.async_copy(pc_hbm.at[pl.ds(i, 1), :, :], pc_b[par], sp),
                pltpu.async_copy(ei_hbm.at[pl.ds(i, 1), :, :], er_b[par], se),
                pltpu.async_copy(u_hbm.at[pl.ds(i, 1), :], u_b[par], su))

    def batch_body(bb, _):
        base = wid * ROWS_PER_TILE + bb * BATCH
        handles = [None, None]
        handles[0] = start_row(base, 0)
        pltpu.sync_copy(zz_hbm, av)
        for t in range(BATCH):
            par = t % 2
            for h in handles[par]:
                h.wait()
            if t + 1 < BATCH:
                handles[(t + 1) % 2] = start_row(base + t + 1, (t + 1) % 2)
            pc_v, er_v, u_v = pc_b[par], er_b[par], u_b[par]
            i_s = jnp.full((16,), base + t, jnp.int32)
            total = plsc.load_gather(pc_v, [zero16, last_hi, last_lo])
            s1 = plsc.load_gather(s12_v, [zero16, i_s])
            logits = []
            neighs = []
            for k in range(8):
                u_k = u_v[0, pl.ds(16 * k, 16)]
                r_k = total * (jnp.float32(1.0) - u_k)
                pos = zero16
                w = 1024
                while w >= 1:
                    cand = pos + w
                    q = cand - 1
                    val = plsc.load_gather(pc_v, [zero16, q >> 7, q & 127])
                    pos = jnp.where(val < r_k, cand, pos)
                    w //= 2
                e_k = plsc.load_gather(er_v, [zero16, pos >> 7, pos & 127])
                s2g = plsc.load_gather(s12_v, [one16, pos])
                s3g = plsc.load_gather(s3_v, [e_k >> 7, e_k & 127])
                z = s1 + s2g + s3g
                logits.append(jnp.where(z >= 0.0, z, jnp.float32(ALPHA) * z))
                neighs.append(pos)
            m = logits[0]
            for k in range(1, 8):
                m = jnp.maximum(m, logits[k])
            mx = lax.reduce_max(m, (0,))
            exps = [jnp.exp(l - mx) for l in logits]
            sv = exps[0]
            for k in range(1, 8):
                sv = sv + exps[k]
            ssum = lax.reduce_sum(sv, (0,))
            t_s = jnp.full((16,), t, jnp.int32)
            for k in range(8):
                att = exps[k] / ssum
                plsc.addupdate_scatter(
                    av, [t_s, neighs[k] >> 7, neighs[k] & 127], att)
        pltpu.sync_copy(av, a_hbm.at[pl.ds(base, BATCH), :, :])
        return 0

    lax.fori_loop(0, NBATCH, batch_body, 0)


def _finish_body(a_ref, x_ref, o_ref):
    i = pl.program_id(0)
    a3 = a_ref[...]
    h = jnp.zeros((256, D), jnp.float32)
    for ct in range(CT):
        h = h + jnp.dot(a3[:, ct, :], x_ref[pl.ds(ct * 128, 128), :],
                        preferred_element_type=jnp.float32)
    xb = x_ref[pl.ds(i * 256, 256), :]
    z = h + xb
    o_ref[...] = jnp.where(z > 0.0, z, jnp.exp(jnp.minimum(z, 0.0)) - 1.0)


@jax.jit
def _run(features, edge_index, edge_emb, W, a):
    a12t = jnp.concatenate(
        [a[:D, 0].reshape(1, D), a[D:2 * D, 0].reshape(1, D),
         jnp.pad(a[2 * D:, 0], (0, D - DE)).reshape(1, D)], axis=0)
    embt = edge_emb.T.reshape(DE, EV // 128, 128)

    x, s12t, s3t, u, rs = pl.pallas_call(
        _prep_body,
        out_shape=(
            jax.ShapeDtypeStruct((N, D), jnp.float32),
            jax.ShapeDtypeStruct((3, N), jnp.float32),
            jax.ShapeDtypeStruct((EV // 128, 128), jnp.float32),
            jax.ShapeDtypeStruct((N, NS), jnp.float32),
            jax.ShapeDtypeStruct((N, 1), jnp.float32),
        ),
    )(edge_index, features, W, a12t, embt)

    p3, e3 = pl.pallas_call(
        _strip_body,
        grid=(8,),
        in_specs=[
            pl.BlockSpec((256, N), lambda i: (i, 0)),
            pl.BlockSpec((256, 1), lambda i: (i, 0)),
        ],
        out_specs=[
            pl.BlockSpec((256, CT, 128), lambda i: (i, 0, 0)),
            pl.BlockSpec((256, CT, 128), lambda i: (i, 0, 0)),
        ],
        out_shape=(
            jax.ShapeDtypeStruct((N, CT, 128), jnp.float32),
            jax.ShapeDtypeStruct((N, CT, 128), jnp.int32),
        ),
    )(edge_index, rs)

    loc = jnp.cumsum(p3, axis=-1)
    tot = loc[..., -1]
    off = jnp.cumsum(tot, axis=-1)
    off_excl = jnp.concatenate(
        [jnp.zeros((N, 1), jnp.float32), off[:, :-1]], axis=1)
    pc3 = loc + off_excl[..., None]

    zz = jnp.zeros((BATCH, CT, 128), jnp.float32)
    mesh = plsc.VectorSubcoreMesh(core_axis_name="c", subcore_axis_name="s")
    amat = pl.kernel(
        _sc_body,
        mesh=mesh,
        compiler_params=pltpu.CompilerParams(needs_layout_passes=False),
        out_type=jax.ShapeDtypeStruct((N, CT, 128), jnp.float32),
        scratch_types=[
            pltpu.VMEM((EV // 128, 128), jnp.float32),
            pltpu.VMEM((2, N), jnp.float32),
            pltpu.VMEM((BATCH, CT, 128), jnp.float32),
            pltpu.VMEM((1, CT, 128), jnp.float32),
            pltpu.VMEM((1, CT, 128), jnp.float32),
            pltpu.VMEM((1, CT, 128), jnp.int32),
            pltpu.VMEM((1, CT, 128), jnp.int32),
            pltpu.VMEM((1, NS), jnp.float32),
            pltpu.VMEM((1, NS), jnp.float32),
            pltpu.SemaphoreType.DMA,
            pltpu.SemaphoreType.DMA,
            pltpu.SemaphoreType.DMA,
            pltpu.SemaphoreType.DMA,
            pltpu.SemaphoreType.DMA,
            pltpu.SemaphoreType.DMA,
        ],
    )(e3, pc3, u, s12t[:2], s3t, zz)

    out = pl.pallas_call(
        _finish_body,
        grid=(8,),
        in_specs=[
            pl.BlockSpec((256, CT, 128), lambda i: (i, 0, 0)),
            pl.BlockSpec((N, D), lambda i: (0, 0)),
        ],
        out_specs=pl.BlockSpec((256, D), lambda i: (i, 0)),
        out_shape=jax.ShapeDtypeStruct((N, D), jnp.float32),
    )(amat, x)
    return out


def kernel(features, index, node_emb, edge_index, edge_emb, n_sample, W, a):
    out = _run(features, edge_index, edge_emb, W, a)
    return (out, edge_emb)


# speedup vs baseline: 11.2224x; 1.0423x over previous
import jax
import jax.numpy as jnp
from jax import lax
from jax.experimental import pallas as pl
from jax.experimental.pallas import tpu as pltpu
from jax.experimental.pallas import tpu_sc as plsc

N = 2048
NS = 128
D = 256
DE = 16
EV = 65536
CT = N // 128
ALPHA = 0.2

NTILES = 32
ROWS_PER_TILE = N // NTILES
BATCH = 16
NBATCH = ROWS_PER_TILE // BATCH


def _threefry2x32(k0, k1, x0, x1):
    ks2 = k0 ^ k1 ^ jnp.uint32(0x1BD11BDA)
    ks = (k0, k1, ks2)
    rot = ((13, 15, 26, 6), (17, 29, 16, 24))
    x0 = x0 + k0
    x1 = x1 + k1
    for i in range(5):
        for r in rot[i % 2]:
            x0 = x0 + x1
            x1 = (x1 << jnp.uint32(r)) | (x1 >> jnp.uint32(32 - r))
            x1 = x0 ^ x1
        x0 = x0 + ks[(i + 1) % 3]
        x1 = x1 + ks[(i + 2) % 3] + jnp.uint32(i + 1)
    return x0, x1


def _prep_body(feat_ref, w_ref, a12t_ref, embt_ref,
               x_ref, s12_ref, s3_ref, u_ref):
    x = jnp.dot(feat_ref[...], w_ref[...], preferred_element_type=jnp.float32)
    x_ref[...] = x
    s12_ref[...] = lax.dot_general(
        a12t_ref[...], x, (((1,), (1,)), ((), ())),
        preferred_element_type=jnp.float32)
    s3 = jnp.zeros((EV // 128, 128), jnp.float32)
    for d in range(DE):
        s3 = s3 + embt_ref[d] * a12t_ref[2, d]
    s3_ref[...] = s3

    zero = jnp.zeros((N, NS), jnp.uint32)
    i_mat = lax.broadcasted_iota(jnp.uint32, (N, NS), 0)
    j_mat = lax.broadcasted_iota(jnp.uint32, (N, NS), 1)
    k0, k1 = _threefry2x32(zero, jnp.full((N, NS), 42, jnp.uint32), zero, i_mat)
    a, b = _threefry2x32(k0, k1, zero, j_mat)
    bits = a ^ b
    fb = (bits >> jnp.uint32(9)) | jnp.uint32(0x3F800000)
    u = lax.bitcast_convert_type(fb, jnp.float32) - jnp.float32(1.0)
    u_ref[...] = jnp.maximum(u, jnp.float32(0.0))


def _strip_body(ei_ref, p3_ref, e3_ref):
    i = pl.program_id(0)
    eib = ei_ref[...]
    mask = (eib != 0).astype(jnp.float32)
    rows = lax.broadcasted_iota(jnp.int32, (256, N), 0) + i * 256
    cols = lax.broadcasted_iota(jnp.int32, (256, N), 1)
    eye = (rows == cols).astype(jnp.float32)
    rs = jnp.sum(mask, axis=1, keepdims=True)
    probs = jnp.where(rs > 0.0, mask / jnp.maximum(rs, 1.0), eye)
    p3_ref[...] = probs.reshape(256, CT, 128)
    e3_ref[...] = eib.reshape(256, CT, 128)


def _sc_body(ei_hbm, lc_hbm, u_hbm, of_hbm, s12_hbm, s3_hbm, zz_hbm,
             a_hbm,
             s3_v, s12_v, av, pcb,
             lc_v0, lc_v1, er_v0, er_v1, u_v0, u_v1, of_v0, of_v1,
             sem_lc0, sem_lc1, sem_er0, sem_er1,
             sem_u0, sem_u1, sem_of0, sem_of1, sem_w, sem_z):
    wid = lax.axis_index("s") * 2 + lax.axis_index("c")
    pltpu.sync_copy(s3_hbm, s3_v)
    pltpu.sync_copy(s12_hbm, s12_v)
    zero16 = jnp.zeros((16,), jnp.int32)
    iota16 = lax.iota(jnp.int32, 16)
    l127 = jnp.full((16,), 127, jnp.int32)
    one16 = jnp.full((16,), 1, jnp.int32)
    f15 = jnp.full((16,), 15, jnp.int32)
    lc_b = (lc_v0, lc_v1)
    er_b = (er_v0, er_v1)
    u_b = (u_v0, u_v1)
    of_b = (of_v0, of_v1)
    sems = ((sem_lc0, sem_er0, sem_u0, sem_of0),
            (sem_lc1, sem_er1, sem_u1, sem_of1))

    def start_row(i, par):
        sl, se, su, so = sems[par]
        return (pltpu.async_copy(lc_hbm.at[pl.ds(i, 1), :, :], lc_b[par], sl),
                pltpu.async_copy(ei_hbm.at[pl.ds(i, 1), :, :], er_b[par], se),
                pltpu.async_copy(u_hbm.at[pl.ds(i, 1), :], u_b[par], su),
                pltpu.async_copy(of_hbm.at[pl.ds(i, 1), :], of_b[par], so))

    def batch_body(bb, _):
        base = wid * ROWS_PER_TILE + bb * BATCH

        @pl.when(bb > 0)
        def _():
            pltpu.make_async_copy(av, a_hbm.at[pl.ds(base, BATCH), :, :],
                                  sem_w).wait()

        handles = [None, None]
        handles[0] = start_row(base, 0)
        hz = pltpu.async_copy(zz_hbm, av, sem_z)
        for t in range(BATCH):
            par = t % 2
            for h in handles[par]:
                h.wait()
            if t + 1 < BATCH:
                handles[(t + 1) % 2] = start_row(base + t + 1, (t + 1) % 2)
            lc_v, er_v, u_v, of_v = lc_b[par], er_b[par], u_b[par], of_b[par]
            i_s = jnp.full((16,), base + t, jnp.int32)
            pcend = plsc.load_gather(lc_v, [zero16, iota16, l127])
            pce = pcend + of_v[0, pl.ds(0, 16)]
            pcb[...] = pce
            total = plsc.load_gather(pcb, [f15])
            s1 = plsc.load_gather(s12_v, [zero16, i_s])
            logits = []
            neighs = []
            for k in range(8):
                u_k = u_v[0, pl.ds(16 * k, 16)]
                r_k = total * (jnp.float32(1.0) - u_k)
                c = zero16
                w = 8
                while w >= 1:
                    cand = c + w
                    val = plsc.load_gather(pcb, [cand - 1])
                    c = jnp.where(val < r_k, cand, c)
                    w //= 2
                off_c = plsc.load_gather(of_v, [zero16, c])
                pos = zero16
                w = 64
                while w >= 1:
                    cand = pos + w
                    val = plsc.load_gather(lc_v, [zero16, c, cand - 1]) + off_c
                    pos = jnp.where(val < r_k, cand, pos)
                    w //= 2
                e_k = plsc.load_gather(er_v, [zero16, c, pos])
                s2g = plsc.load_gather(s12_v, [one16, (c << 7) + pos])
                s3g = plsc.load_gather(s3_v, [e_k >> 7, e_k & 127])
                z = s1 + s2g + s3g
                logits.append(jnp.where(z >= 0.0, z, jnp.float32(ALPHA) * z))
                neighs.append((c, pos))
            m = logits[0]
            for k in range(1, 8):
                m = jnp.maximum(m, logits[k])
            mx = lax.reduce_max(m, (0,))
            exps = [jnp.exp(l - mx) for l in logits]
            sv = exps[0]
            for k in range(1, 8):
                sv = sv + exps[k]
            ssum = lax.reduce_sum(sv, (0,))
            if t == 0:
                hz.wait()
            t_s = jnp.full((16,), t, jnp.int32)
            for k in range(8):
                att = exps[k] / ssum
                c, pos = neighs[k]
                plsc.addupdate_scatter(av, [t_s, c, pos], att)
        pltpu.async_copy(av, a_hbm.at[pl.ds(base, BATCH), :, :], sem_w)
        return 0

    lax.fori_loop(0, NBATCH, batch_body, 0)
    pltpu.make_async_copy(av, a_hbm.at[pl.ds(0, BATCH), :, :], sem_w).wait()


def _finish_body(a_ref, x_ref, o_ref):
    i = pl.program_id(0)
    a3 = a_ref[...]
    h = jnp.zeros((256, D), jnp.float32)
    for ct in range(CT):
        h = h + jnp.dot(a3[:, ct, :], x_ref[pl.ds(ct * 128, 128), :],
                        preferred_element_type=jnp.float32)
    xb = x_ref[pl.ds(i * 256, 256), :]
    z = h + xb
    o_ref[...] = jnp.where(z > 0.0, z, jnp.exp(jnp.minimum(z, 0.0)) - 1.0)


@jax.jit
def _run(features, edge_index, edge_emb, W, a):
    a12t = jnp.concatenate(
        [a[:D, 0].reshape(1, D), a[D:2 * D, 0].reshape(1, D),
         jnp.pad(a[2 * D:, 0], (0, D - DE)).reshape(1, D)], axis=0)
    embt = edge_emb.T.reshape(DE, EV // 128, 128)

    x, s12t, s3t, u = pl.pallas_call(
        _prep_body,
        out_shape=(
            jax.ShapeDtypeStruct((N, D), jnp.float32),
            jax.ShapeDtypeStruct((3, N), jnp.float32),
            jax.ShapeDtypeStruct((EV // 128, 128), jnp.float32),
            jax.ShapeDtypeStruct((N, NS), jnp.float32),
        ),
    )(features, W, a12t, embt)

    p3, e3 = pl.pallas_call(
        _strip_body,
        grid=(8,),
        in_specs=[
            pl.BlockSpec((256, N), lambda i: (i, 0)),
        ],
        out_specs=[
            pl.BlockSpec((256, CT, 128), lambda i: (i, 0, 0)),
            pl.BlockSpec((256, CT, 128), lambda i: (i, 0, 0)),
        ],
        out_shape=(
            jax.ShapeDtypeStruct((N, CT, 128), jnp.float32),
            jax.ShapeDtypeStruct((N, CT, 128), jnp.int32),
        ),
    )(edge_index)

    loc = jnp.cumsum(p3, axis=-1)
    tot = loc[..., -1]
    off = jnp.cumsum(tot, axis=-1)
    off_excl = jnp.concatenate(
        [jnp.zeros((N, 1), jnp.float32), off[:, :-1]], axis=1)
    off_pad = jnp.pad(off_excl, ((0, 0), (0, 128 - CT)))

    zz = jnp.zeros((BATCH, CT, 128), jnp.float32)
    mesh = plsc.VectorSubcoreMesh(core_axis_name="c", subcore_axis_name="s")
    amat = pl.kernel(
        _sc_body,
        mesh=mesh,
        compiler_params=pltpu.CompilerParams(needs_layout_passes=False),
        out_type=jax.ShapeDtypeStruct((N, CT, 128), jnp.float32),
        scratch_types=[
            pltpu.VMEM((EV // 128, 128), jnp.float32),
            pltpu.VMEM((2, N), jnp.float32),
            pltpu.VMEM((BATCH, CT, 128), jnp.float32),
            pltpu.VMEM((16,), jnp.float32),
            pltpu.VMEM((1, CT, 128), jnp.float32),
            pltpu.VMEM((1, CT, 128), jnp.float32),
            pltpu.VMEM((1, CT, 128), jnp.int32),
            pltpu.VMEM((1, CT, 128), jnp.int32),
            pltpu.VMEM((1, NS), jnp.float32),
            pltpu.VMEM((1, NS), jnp.float32),
            pltpu.VMEM((1, NS), jnp.float32),
            pltpu.VMEM((1, NS), jnp.float32),
            pltpu.SemaphoreType.DMA,
            pltpu.SemaphoreType.DMA,
            pltpu.SemaphoreType.DMA,
            pltpu.SemaphoreType.DMA,
            pltpu.SemaphoreType.DMA,
            pltpu.SemaphoreType.DMA,
            pltpu.SemaphoreType.DMA,
            pltpu.SemaphoreType.DMA,
            pltpu.SemaphoreType.DMA,
            pltpu.SemaphoreType.DMA,
        ],
    )(e3, loc, u, off_pad, s12t[:2], s3t, zz)

    out = pl.pallas_call(
        _finish_body,
        grid=(8,),
        in_specs=[
            pl.BlockSpec((256, CT, 128), lambda i: (i, 0, 0)),
            pl.BlockSpec((N, D), lambda i: (0, 0)),
        ],
        out_specs=pl.BlockSpec((256, D), lambda i: (i, 0)),
        out_shape=jax.ShapeDtypeStruct((N, D), jnp.float32),
    )(amat, x)
    return out


def kernel(features, index, node_emb, edge_index, edge_emb, n_sample, W, a):
    out = _run(features, edge_index, edge_emb, W, a)
    return (out, edge_emb)


# speedup vs baseline: 11.9063x; 1.0609x over previous
import jax
import jax.numpy as jnp
from jax import lax
from jax.experimental import pallas as pl
from jax.experimental.pallas import tpu as pltpu
from jax.experimental.pallas import tpu_sc as plsc

N = 2048
NS = 128
D = 256
DE = 16
EV = 65536
CT = N // 128
ALPHA = 0.2

NTILES = 32
ROWS_PER_TILE = N // NTILES
BATCH = 16
NBATCH = ROWS_PER_TILE // BATCH


def _threefry2x32(k0, k1, x0, x1):
    ks2 = k0 ^ k1 ^ jnp.uint32(0x1BD11BDA)
    ks = (k0, k1, ks2)
    rot = ((13, 15, 26, 6), (17, 29, 16, 24))
    x0 = x0 + k0
    x1 = x1 + k1
    for i in range(5):
        for r in rot[i % 2]:
            x0 = x0 + x1
            x1 = (x1 << jnp.uint32(r)) | (x1 >> jnp.uint32(32 - r))
            x1 = x0 ^ x1
        x0 = x0 + ks[(i + 1) % 3]
        x1 = x1 + ks[(i + 2) % 3] + jnp.uint32(i + 1)
    return x0, x1


def _prep_body(feat_ref, w_ref, a12t_ref, embt_ref,
               x_ref, s12_ref, s3_ref, u_ref):
    x = jnp.dot(feat_ref[...], w_ref[...], preferred_element_type=jnp.float32)
    x_ref[...] = x
    s12_ref[...] = lax.dot_general(
        a12t_ref[...][:2], x, (((1,), (1,)), ((), ())),
        preferred_element_type=jnp.float32)
    s3 = jnp.zeros((EV // 128, 128), jnp.float32)
    for d in range(DE):
        s3 = s3 + embt_ref[d] * a12t_ref[2, d]
    s3_ref[...] = s3

    zero = jnp.zeros((N, NS), jnp.uint32)
    i_mat = lax.broadcasted_iota(jnp.uint32, (N, NS), 0)
    j_mat = lax.broadcasted_iota(jnp.uint32, (N, NS), 1)
    k0, k1 = _threefry2x32(zero, jnp.full((N, NS), 42, jnp.uint32), zero, i_mat)
    a, b = _threefry2x32(k0, k1, zero, j_mat)
    bits = a ^ b
    fb = (bits >> jnp.uint32(9)) | jnp.uint32(0x3F800000)
    u = lax.bitcast_convert_type(fb, jnp.float32) - jnp.float32(1.0)
    u_ref[...] = jnp.maximum(u, jnp.float32(0.0))


def _strip_body(ei_ref, p3_ref, e3_ref):
    i = pl.program_id(0)
    eib = ei_ref[...]
    mask = (eib != 0).astype(jnp.float32)
    rows = lax.broadcasted_iota(jnp.int32, (256, N), 0) + i * 256
    cols = lax.broadcasted_iota(jnp.int32, (256, N), 1)
    eye = (rows == cols).astype(jnp.float32)
    rs = jnp.sum(mask, axis=1, keepdims=True)
    probs = jnp.where(rs > 0.0, mask / jnp.maximum(rs, 1.0), eye)
    p3_ref[...] = probs.reshape(256, CT, 128)
    e3_ref[...] = eib.reshape(256, CT, 128)


def _sc_body(ei_hbm, lc_hbm, u_hbm, of_hbm, s12_hbm, s3_hbm, zz_hbm,
             a_hbm,
             s3_v, s12_v, av, pcb,
             lc_v0, lc_v1, er_v0, er_v1, u_v0, u_v1, of_v0, of_v1,
             sem_lc0, sem_lc1, sem_er0, sem_er1,
             sem_u0, sem_u1, sem_of0, sem_of1, sem_w, sem_z):
    wid = lax.axis_index("s") * 2 + lax.axis_index("c")
    pltpu.sync_copy(s3_hbm, s3_v)
    pltpu.sync_copy(s12_hbm, s12_v)
    zero16 = jnp.zeros((16,), jnp.int32)
    iota16 = lax.iota(jnp.int32, 16)
    l127 = jnp.full((16,), 127, jnp.int32)
    one16 = jnp.full((16,), 1, jnp.int32)
    f15 = jnp.full((16,), 15, jnp.int32)
    lc_b = (lc_v0, lc_v1)
    er_b = (er_v0, er_v1)
    u_b = (u_v0, u_v1)
    of_b = (of_v0, of_v1)
    sems = ((sem_lc0, sem_er0, sem_u0, sem_of0),
            (sem_lc1, sem_er1, sem_u1, sem_of1))

    def start_pair(i, par):
        sl, se, su, so = sems[par]
        return (pltpu.async_copy(lc_hbm.at[pl.ds(i, 2), :, :], lc_b[par], sl),
                pltpu.async_copy(ei_hbm.at[pl.ds(i, 2), :, :], er_b[par], se),
                pltpu.async_copy(u_hbm.at[pl.ds(i, 2), :], u_b[par], su),
                pltpu.async_copy(of_hbm.at[pl.ds(i, 2), :], of_b[par], so))

    def batch_body(bb, _):
        base = wid * ROWS_PER_TILE + bb * BATCH

        @pl.when(bb > 0)
        def _():
            pltpu.make_async_copy(av, a_hbm.at[pl.ds(base, BATCH), :, :],
                                  sem_w).wait()

        handles = [None, None]
        handles[0] = start_pair(base, 0)
        hz = pltpu.async_copy(zz_hbm, av, sem_z)
        for p in range(BATCH // 2):
            par = p % 2
            for h in handles[par]:
                h.wait()
            if p + 1 < BATCH // 2:
                handles[(p + 1) % 2] = start_pair(base + 2 * (p + 1),
                                                  (p + 1) % 2)
            lc_v, er_v, u_v, of_v = lc_b[par], er_b[par], u_b[par], of_b[par]
            rows = []
            for rr in range(2):
                t = 2 * p + rr
                rr16 = jnp.full((16,), rr, jnp.int32)
                i_s = jnp.full((16,), base + t, jnp.int32)
                pcend = plsc.load_gather(lc_v, [rr16, iota16, l127])
                pce = pcend + of_v[rr, pl.ds(0, 16)]
                pcb[pl.ds(16 * rr, 16)] = pce
                total = plsc.load_gather(pcb, [f15 + 16 * rr])
                s1 = plsc.load_gather(s12_v, [zero16, i_s])
                rows.append((t, rr, rr16, total, s1))
            outs = []
            for (t, rr, rr16, total, s1) in rows:
                logits = []
                neighs = []
                for k in range(8):
                    u_k = u_v[rr, pl.ds(16 * k, 16)]
                    r_k = total * (jnp.float32(1.0) - u_k)
                    c = zero16
                    w = 8
                    while w >= 1:
                        cand = c + w
                        val = plsc.load_gather(pcb, [cand - 1 + 16 * rr])
                        c = jnp.where(val < r_k, cand, c)
                        w //= 2
                    off_c = plsc.load_gather(of_v, [rr16, c])
                    pos = zero16
                    w = 64
                    while w >= 1:
                        cand = pos + w
                        val = plsc.load_gather(
                            lc_v, [rr16, c, cand - 1]) + off_c
                        pos = jnp.where(val < r_k, cand, pos)
                        w //= 2
                    e_k = plsc.load_gather(er_v, [rr16, c, pos])
                    s2g = plsc.load_gather(s12_v, [one16, (c << 7) + pos])
                    s3g = plsc.load_gather(s3_v, [e_k >> 7, e_k & 127])
                    z = s1 + s2g + s3g
                    logits.append(
                        jnp.where(z >= 0.0, z, jnp.float32(ALPHA) * z))
                    neighs.append((c, pos))
                outs.append((t, logits, neighs))
            if p == 0:
                hz.wait()
            for (t, logits, neighs) in outs:
                m = logits[0]
                for k in range(1, 8):
                    m = jnp.maximum(m, logits[k])
                mx = lax.reduce_max(m, (0,))
                exps = [jnp.exp(l - mx) for l in logits]
                sv = exps[0]
                for k in range(1, 8):
                    sv = sv + exps[k]
                ssum = lax.reduce_sum(sv, (0,))
                t_s = jnp.full((16,), t, jnp.int32)
                for k in range(8):
                    att = exps[k] / ssum
                    c, pos = neighs[k]
                    plsc.addupdate_scatter(av, [t_s, c, pos], att)
        pltpu.async_copy(av, a_hbm.at[pl.ds(base, BATCH), :, :], sem_w)
        return 0

    lax.fori_loop(0, NBATCH, batch_body, 0)
    pltpu.make_async_copy(av, a_hbm.at[pl.ds(0, BATCH), :, :], sem_w).wait()


def _finish_body(a_ref, x_ref, o_ref):
    i = pl.program_id(0)
    a3 = a_ref[...]
    h = jnp.zeros((256, D), jnp.float32)
    for ct in range(CT):
        h = h + jnp.dot(a3[:, ct, :], x_ref[pl.ds(ct * 128, 128), :],
                        preferred_element_type=jnp.float32)
    xb = x_ref[pl.ds(i * 256, 256), :]
    z = h + xb
    o_ref[...] = jnp.where(z > 0.0, z, jnp.exp(jnp.minimum(z, 0.0)) - 1.0)


@jax.jit
def _run(features, edge_index, edge_emb, W, a):
    a12t = jnp.concatenate(
        [a[:D, 0].reshape(1, D), a[D:2 * D, 0].reshape(1, D),
         jnp.pad(a[2 * D:, 0], (0, D - DE)).reshape(1, D)], axis=0)
    embt = edge_emb.T.reshape(DE, EV // 128, 128)

    x, s12t, s3t, u = pl.pallas_call(
        _prep_body,
        out_shape=(
            jax.ShapeDtypeStruct((N, D), jnp.float32),
            jax.ShapeDtypeStruct((2, N), jnp.float32),
            jax.ShapeDtypeStruct((EV // 128, 128), jnp.float32),
            jax.ShapeDtypeStruct((N, NS), jnp.float32),
        ),
    )(features, W, a12t, embt)

    p3, e3 = pl.pallas_call(
        _strip_body,
        grid=(8,),
        in_specs=[
            pl.BlockSpec((256, N), lambda i: (i, 0)),
        ],
        out_specs=[
            pl.BlockSpec((256, CT, 128), lambda i: (i, 0, 0)),
            pl.BlockSpec((256, CT, 128), lambda i: (i, 0, 0)),
        ],
        out_shape=(
            jax.ShapeDtypeStruct((N, CT, 128), jnp.float32),
            jax.ShapeDtypeStruct((N, CT, 128), jnp.int32),
        ),
    )(edge_index)

    loc = jnp.cumsum(p3, axis=-1)
    tot = loc[..., -1]
    off = jnp.cumsum(tot, axis=-1)
    off_excl = jnp.concatenate(
        [jnp.zeros((N, 1), jnp.float32), off[:, :-1]], axis=1)
    off_pad = jnp.pad(off_excl, ((0, 0), (0, 128 - CT)))

    zz = jnp.zeros((BATCH, CT, 128), jnp.float32)
    mesh = plsc.VectorSubcoreMesh(core_axis_name="c", subcore_axis_name="s")
    amat = pl.kernel(
        _sc_body,
        mesh=mesh,
        compiler_params=pltpu.CompilerParams(needs_layout_passes=False),
        out_type=jax.ShapeDtypeStruct((N, CT, 128), jnp.float32),
        scratch_types=[
            pltpu.VMEM((EV // 128, 128), jnp.float32),
            pltpu.VMEM((2, N), jnp.float32),
            pltpu.VMEM((BATCH, CT, 128), jnp.float32),
            pltpu.VMEM((32,), jnp.float32),
            pltpu.VMEM((2, CT, 128), jnp.float32),
            pltpu.VMEM((2, CT, 128), jnp.float32),
            pltpu.VMEM((2, CT, 128), jnp.int32),
            pltpu.VMEM((2, CT, 128), jnp.int32),
            pltpu.VMEM((2, NS), jnp.float32),
            pltpu.VMEM((2, NS), jnp.float32),
            pltpu.VMEM((2, NS), jnp.float32),
            pltpu.VMEM((2, NS), jnp.float32),
            pltpu.SemaphoreType.DMA,
            pltpu.SemaphoreType.DMA,
            pltpu.SemaphoreType.DMA,
            pltpu.SemaphoreType.DMA,
            pltpu.SemaphoreType.DMA,
            pltpu.SemaphoreType.DMA,
            pltpu.SemaphoreType.DMA,
            pltpu.SemaphoreType.DMA,
            pltpu.SemaphoreType.DMA,
            pltpu.SemaphoreType.DMA,
        ],
    )(e3, loc, u, off_pad, s12t, s3t, zz)

    out = pl.pallas_call(
        _finish_body,
        grid=(8,),
        in_specs=[
            pl.BlockSpec((256, CT, 128), lambda i: (i, 0, 0)),
            pl.BlockSpec((N, D), lambda i: (0, 0)),
        ],
        out_specs=pl.BlockSpec((256, D), lambda i: (i, 0)),
        out_shape=jax.ShapeDtypeStruct((N, D), jnp.float32),
    )(amat, x)
    return out


def kernel(features, index, node_emb, edge_index, edge_emb, n_sample, W, a):
    out = _run(features, edge_index, edge_emb, W, a)
    return (out, edge_emb)
